# baseline (speedup 1.0000x reference)
import functools

import jax
import jax.numpy as jnp
from jax import lax
from jax.experimental import pallas as pl
from jax.experimental.pallas import tpu as pltpu
from jax.experimental.pallas import tpu_sc as plsc

_GAMMA = 12.0
_B = 16384
_D = 64
_NC = 2
_NS = 16
_NW = _NC * _NS
_BPW = _B // _NW
_C = 128
_NCHUNK = _BPW // _C
_L = 16


def _make_sc_kernel():
    mesh = plsc.VectorSubcoreMesh(core_axis_name="c", subcore_axis_name="s")

    @functools.partial(
        pl.kernel,
        mesh=mesh,
        compiler_params=pltpu.CompilerParams(
            needs_layout_passes=False, use_tc_tiling_on_sc=False),
        out_type=jax.ShapeDtypeStruct((_NW, _BPW), jnp.float32),
        scratch_types=[
            pltpu.VMEM((_NCHUNK, _C), jnp.int32),
            pltpu.VMEM((_NCHUNK, _C), jnp.int32),
            pltpu.VMEM((_NCHUNK, _C), jnp.int32),
            pltpu.VMEM((_C, _D), jnp.float32),
            pltpu.VMEM((_C, _D), jnp.float32),
            pltpu.VMEM((_C, _D), jnp.float32),
            pltpu.VMEM((_BPW,), jnp.float32),
            pltpu.VMEM((_L, _L + 1), jnp.float32),
            pltpu.SemaphoreType.DMA,
        ],
    )
    def sc_kernel(hidx, ridx, tidx, ent, rel, out,
                  hi_v, ri_v, ti_v, hbuf, rbuf, tbuf, out_v, tr, sem):
        wid = lax.axis_index("s") * _NC + lax.axis_index("c")
        pltpu.sync_copy(hidx.at[wid], hi_v)
        pltpu.sync_copy(ridx.at[wid], ri_v)
        pltpu.sync_copy(tidx.at[wid], ti_v)
        lane = lax.iota(jnp.int32, _L)
        for k in range(_NCHUNK):
            c1 = pltpu.async_copy(ent.at[hi_v.at[k]], hbuf, sem)
            c2 = pltpu.async_copy(rel.at[ri_v.at[k]], rbuf, sem)
            c3 = pltpu.async_copy(ent.at[ti_v.at[k]], tbuf, sem)
            c1.wait()
            c2.wait()
            c3.wait()

            def group(g, carry, k=k):
                for i in range(_L):
                    row = g * _L + i
                    acc = jnp.zeros((_L,), jnp.float32)
                    for q in range(_D // _L):
                        h = hbuf[row, pl.ds(q * _L, _L)]
                        r = rbuf[row, pl.ds(q * _L, _L)]
                        t = tbuf[row, pl.ds(q * _L, _L)]
                        acc = acc + jnp.abs(h + r - t)
                    tr[i, pl.ds(0, _L)] = acc
                res = jnp.zeros((_L,), jnp.float32)
                for c in range(_L):
                    col = jnp.full((_L,), c, jnp.int32)
                    res = res + plsc.load_gather(tr, [lane, col])
                out_v[pl.ds(k * _C + g * _L, _L)] = _GAMMA - res
                return carry

            lax.fori_loop(0, _C // _L, group, 0)
        pltpu.sync_copy(out_v, out.at[wid])

    return sc_kernel


_sc_kernel = _make_sc_kernel()


def kernel(sample, entity_embedding, relation_embedding):
    hidx = sample[:, 0].reshape(_NW, _NCHUNK, _C)
    ridx = sample[:, 1].reshape(_NW, _NCHUNK, _C)
    tidx = sample[:, 2].reshape(_NW, _NCHUNK, _C)
    out = _sc_kernel(hidx, ridx, tidx, entity_embedding, relation_embedding)
    return out.reshape(_B, 1)
